# baseline (speedup 1.0000x reference)
import functools

import jax
import jax.numpy as jnp
from jax import lax
from jax.experimental import pallas as pl
from jax.experimental.pallas import tpu as pltpu
from jax.experimental.pallas import tpu_sc as plsc

_N = 10000
_E = 320000
_D = 128
_NC = 2
_NS = 16
_NW = _NC * _NS
_LANES = 96


def _sc_segment_sum(h, pk0, pk1, zeros):
    K0 = pk0.shape[1]
    K1 = pk1.shape[1]
    assert K0 % 2 == 1 and K1 % 2 == 1, "pipeline assumes odd chunk counts"
    ACC = zeros.shape[0]
    mesh = plsc.VectorSubcoreMesh(core_axis_name="c", subcore_axis_name="s")

    @functools.partial(
        pl.kernel,
        out_type=jax.ShapeDtypeStruct((_NC, ACC, _D), jnp.float32),
        mesh=mesh,
        scratch_types=[
            pltpu.VMEM((max(K0, K1), _LANES), jnp.int32),
            pltpu.VMEM((2, _LANES), jnp.int32),
            pltpu.VMEM((2, _LANES), jnp.int32),
            pltpu.VMEM((_LANES, _D), jnp.float32),
            pltpu.VMEM((_LANES, _D), jnp.float32),
            pltpu.VMEM_SHARED((ACC, _D), jnp.float32),
            pltpu.SemaphoreType.DMA,
            pltpu.SemaphoreType.DMA,
        ],
    )
    def k(h_hbm, pk0_hbm, pk1_hbm, z_hbm, out_hbm, pk_v, idx_a, idx_b,
          rows_a, rows_b, acc, sem_a, sem_b):
        c = lax.axis_index("c")
        s = lax.axis_index("s")
        zrows = ACC // _NS
        pltpu.sync_copy(z_hbm.at[pl.ds(s * zrows, zrows)],
                        acc.at[pl.ds(s * zrows, zrows)])
        plsc.subcore_barrier()

        def unpack(t, idx):
            row = pk_v.at[t]
            for j in range(0, _LANES, 16):
                v = row[pl.ds(j, 16)]
                idx.at[0][pl.ds(j, 16)] = v & 0xFFFF
                idx.at[1][pl.ds(j, 16)] = v >> 16

        def run(pk_hbm, K):
            pltpu.sync_copy(pk_hbm.at[s], pk_v.at[pl.ds(0, K)])
            unpack(0, idx_a)
            pltpu.async_copy(h_hbm.at[idx_a.at[0]], rows_a, sem_a)

            @pl.loop(0, K - 1, step=2)
            def _(t):
                unpack(t + 1, idx_b)
                pltpu.async_copy(h_hbm.at[idx_b.at[0]], rows_b, sem_b)
                pltpu.make_async_copy(h_hbm.at[idx_a.at[0]], rows_a,
                                      sem_a).wait()
                pltpu.sync_copy(rows_a, acc.at[idx_a.at[1]], add=True)
                unpack(t + 2, idx_a)
                pltpu.async_copy(h_hbm.at[idx_a.at[0]], rows_a, sem_a)
                pltpu.make_async_copy(h_hbm.at[idx_b.at[0]], rows_b,
                                      sem_b).wait()
                pltpu.sync_copy(rows_b, acc.at[idx_b.at[1]], add=True)

            pltpu.make_async_copy(h_hbm.at[idx_a.at[0]], rows_a, sem_a).wait()
            pltpu.sync_copy(rows_a, acc.at[idx_a.at[1]], add=True)

        @pl.when(c == 0)
        def _():
            run(pk0_hbm, K0)

        @pl.when(c == 1)
        def _():
            run(pk1_hbm, K1)

        plsc.subcore_barrier()
        pltpu.sync_copy(acc.at[pl.ds(s * zrows, zrows)],
                        out_hbm.at[c, pl.ds(s * zrows, zrows)])

    return k(h, pk0, pk1, zeros)


def _tc_dense(h, p, deg2, W_self, W_neigh, bias2):
    B = 1000
    dn = (((1,), (1,)), ((), ()))

    def body(h_ref, p0_ref, p1_ref, deg_ref, ws_ref, wn_ref, b_ref, o_ref):
        a = (p0_ref[0] + p1_ref[0]) / deg_ref[...]
        acc = lax.dot_general(h_ref[...], ws_ref[...], dn,
                              preferred_element_type=jnp.float32,
                              precision=lax.Precision.HIGHEST)
        acc = acc + lax.dot_general(a, wn_ref[...], dn,
                                    preferred_element_type=jnp.float32,
                                    precision=lax.Precision.HIGHEST)
        o_ref[...] = jnp.maximum(acc + b_ref[...], 0.0)

    return pl.pallas_call(
        body,
        grid=(_N // B,),
        in_specs=[
            pl.BlockSpec((B, _D), lambda i: (i, 0)),
            pl.BlockSpec((1, B, _D), lambda i: (0, i, 0)),
            pl.BlockSpec((1, B, _D), lambda i: (1, i, 0)),
            pl.BlockSpec((B, 1), lambda i: (i, 0)),
            pl.BlockSpec((_D, _D), lambda i: (0, 0)),
            pl.BlockSpec((_D, _D), lambda i: (0, 0)),
            pl.BlockSpec((1, _D), lambda i: (0, 0)),
        ],
        out_specs=pl.BlockSpec((B, _D), lambda i: (i, 0)),
        out_shape=jax.ShapeDtypeStruct((_N, _D), jnp.float32),
    )(h, p, p, deg2, W_self, W_neigh, bias2)


def kernel(h, edge_index, deg, W_self, W_neigh, bias):
    src = edge_index[0]
    dst = edge_index[1]
    K0, K1 = 181, 29
    e0 = _NS * K0 * _LANES
    e1_pad = _NS * K1 * _LANES
    assert e0 + e1_pad >= _E
    acc_rows = 128 * (-(-(_N + 1) // 128))
    packed = src | (dst << 16)
    pad = jnp.full((e0 + e1_pad - _E,), _N << 16, jnp.int32)
    pk0 = packed[:e0].reshape(_NS, K0, _LANES)
    pk1 = jnp.concatenate([packed[e0:], pad]).reshape(_NS, K1, _LANES)
    zeros = jnp.zeros((acc_rows, _D), jnp.float32)
    partials = _sc_segment_sum(h, pk0, pk1, zeros)
    return _tc_dense(h, partials, deg.reshape(_N, 1), W_self, W_neigh,
                     bias.reshape(1, _D))


# speedup vs baseline: 1.0311x; 1.0311x over previous
import functools

import jax
import jax.numpy as jnp
from jax import lax
from jax.experimental import pallas as pl
from jax.experimental.pallas import tpu as pltpu
from jax.experimental.pallas import tpu_sc as plsc

_N = 10000
_E = 320000
_D = 128
_NC = 2
_NS = 16
_NW = _NC * _NS
_LANES = 112


def _sc_segment_sum(h, pk0, pk1, zeros):
    K0 = pk0.shape[1]
    K1 = pk1.shape[1]
    assert K0 % 2 == 1 and K1 % 2 == 1, "pipeline assumes odd chunk counts"
    ACC = zeros.shape[0]
    mesh = plsc.VectorSubcoreMesh(core_axis_name="c", subcore_axis_name="s")

    @functools.partial(
        pl.kernel,
        out_type=jax.ShapeDtypeStruct((_NC, ACC, _D), jnp.float32),
        mesh=mesh,
        scratch_types=[
            pltpu.VMEM((max(K0, K1), _LANES), jnp.int32),
            pltpu.VMEM((2, _LANES), jnp.int32),
            pltpu.VMEM((2, _LANES), jnp.int32),
            pltpu.VMEM((_LANES, _D), jnp.float32),
            pltpu.VMEM((_LANES, _D), jnp.float32),
            pltpu.VMEM_SHARED((ACC, _D), jnp.float32),
            pltpu.SemaphoreType.DMA,
            pltpu.SemaphoreType.DMA,
        ],
    )
    def k(h_hbm, pk0_hbm, pk1_hbm, z_hbm, out_hbm, pk_v, idx_a, idx_b,
          rows_a, rows_b, acc, sem_a, sem_b):
        c = lax.axis_index("c")
        s = lax.axis_index("s")
        zrows = ACC // _NS
        pltpu.sync_copy(z_hbm.at[pl.ds(s * zrows, zrows)],
                        acc.at[pl.ds(s * zrows, zrows)])
        plsc.subcore_barrier()

        def unpack(t, idx):
            row = pk_v.at[t]
            for j in range(0, _LANES, 16):
                v = row[pl.ds(j, 16)]
                idx.at[0][pl.ds(j, 16)] = v & 0xFFFF
                idx.at[1][pl.ds(j, 16)] = v >> 16

        def run(pk_hbm, K):
            pltpu.sync_copy(pk_hbm.at[s], pk_v.at[pl.ds(0, K)])
            unpack(0, idx_a)
            pltpu.async_copy(h_hbm.at[idx_a.at[0]], rows_a, sem_a)

            @pl.loop(0, K - 1, step=2)
            def _(t):
                unpack(t + 1, idx_b)
                pltpu.async_copy(h_hbm.at[idx_b.at[0]], rows_b, sem_b)
                pltpu.make_async_copy(h_hbm.at[idx_a.at[0]], rows_a,
                                      sem_a).wait()
                pltpu.sync_copy(rows_a, acc.at[idx_a.at[1]], add=True)
                unpack(t + 2, idx_a)
                pltpu.async_copy(h_hbm.at[idx_a.at[0]], rows_a, sem_a)
                pltpu.make_async_copy(h_hbm.at[idx_b.at[0]], rows_b,
                                      sem_b).wait()
                pltpu.sync_copy(rows_b, acc.at[idx_b.at[1]], add=True)

            pltpu.make_async_copy(h_hbm.at[idx_a.at[0]], rows_a, sem_a).wait()
            pltpu.sync_copy(rows_a, acc.at[idx_a.at[1]], add=True)

        @pl.when(c == 0)
        def _():
            run(pk0_hbm, K0)

        @pl.when(c == 1)
        def _():
            run(pk1_hbm, K1)

        plsc.subcore_barrier()
        pltpu.sync_copy(acc.at[pl.ds(s * zrows, zrows)],
                        out_hbm.at[c, pl.ds(s * zrows, zrows)])

    return k(h, pk0, pk1, zeros)


def _tc_dense(h, p, deg2, W_self, W_neigh, bias2):
    B = 1000
    dn = (((1,), (1,)), ((), ()))

    def body(h_ref, p0_ref, p1_ref, deg_ref, ws_ref, wn_ref, b_ref, o_ref):
        a = (p0_ref[0] + p1_ref[0]) / deg_ref[...]
        acc = lax.dot_general(h_ref[...], ws_ref[...], dn,
                              preferred_element_type=jnp.float32,
                              precision=lax.Precision.HIGHEST)
        acc = acc + lax.dot_general(a, wn_ref[...], dn,
                                    preferred_element_type=jnp.float32,
                                    precision=lax.Precision.HIGHEST)
        o_ref[...] = jnp.maximum(acc + b_ref[...], 0.0)

    return pl.pallas_call(
        body,
        grid=(_N // B,),
        in_specs=[
            pl.BlockSpec((B, _D), lambda i: (i, 0)),
            pl.BlockSpec((1, B, _D), lambda i: (0, i, 0)),
            pl.BlockSpec((1, B, _D), lambda i: (1, i, 0)),
            pl.BlockSpec((B, 1), lambda i: (i, 0)),
            pl.BlockSpec((_D, _D), lambda i: (0, 0)),
            pl.BlockSpec((_D, _D), lambda i: (0, 0)),
            pl.BlockSpec((1, _D), lambda i: (0, 0)),
        ],
        out_specs=pl.BlockSpec((B, _D), lambda i: (i, 0)),
        out_shape=jax.ShapeDtypeStruct((_N, _D), jnp.float32),
    )(h, p, p, deg2, W_self, W_neigh, bias2)


def kernel(h, edge_index, deg, W_self, W_neigh, bias):
    src = edge_index[0]
    dst = edge_index[1]
    K0, K1 = 147, 33
    e0 = _NS * K0 * _LANES
    e1_pad = _NS * K1 * _LANES
    assert e0 + e1_pad >= _E
    acc_rows = 128 * (-(-(_N + 1) // 128))
    packed = src | (dst << 16)
    pad = jnp.full((e0 + e1_pad - _E,), _N << 16, jnp.int32)
    pk0 = packed[:e0].reshape(_NS, K0, _LANES)
    pk1 = jnp.concatenate([packed[e0:], pad]).reshape(_NS, K1, _LANES)
    zeros = jnp.zeros((acc_rows, _D), jnp.float32)
    partials = _sc_segment_sum(h, pk0, pk1, zeros)
    return _tc_dense(h, partials, deg.reshape(_N, 1), W_self, W_neigh,
                     bias.reshape(1, _D))


# speedup vs baseline: 1.0473x; 1.0157x over previous
import functools

import jax
import jax.numpy as jnp
from jax import lax
from jax.experimental import pallas as pl
from jax.experimental.pallas import tpu as pltpu
from jax.experimental.pallas import tpu_sc as plsc

_N = 10000
_E = 320000
_D = 128
_NC = 2
_NS = 16
_NW = _NC * _NS
_LANES = 112


def _sc_segment_sum(h, pk0, pk1, zeros):
    K0 = pk0.shape[1]
    K1 = pk1.shape[1]
    assert K0 % 2 == 1 and K1 % 2 == 1, "pipeline assumes odd chunk counts"
    ACC = zeros.shape[0]
    mesh = plsc.VectorSubcoreMesh(core_axis_name="c", subcore_axis_name="s")

    @functools.partial(
        pl.kernel,
        out_type=jax.ShapeDtypeStruct((_NC, ACC, _D), jnp.float32),
        mesh=mesh,
        scratch_types=[
            pltpu.VMEM((max(K0, K1), _LANES), jnp.int32),
            pltpu.VMEM((2, _LANES), jnp.int32),
            pltpu.VMEM((2, _LANES), jnp.int32),
            pltpu.VMEM((_LANES, _D), jnp.float32),
            pltpu.VMEM((_LANES, _D), jnp.float32),
            pltpu.VMEM_SHARED((ACC, _D), jnp.float32),
            pltpu.SemaphoreType.DMA,
            pltpu.SemaphoreType.DMA,
        ],
    )
    def k(h_hbm, pk0_hbm, pk1_hbm, z_hbm, out_hbm, pk_v, idx_a, idx_b,
          rows_a, rows_b, acc, sem_a, sem_b):
        c = lax.axis_index("c")
        s = lax.axis_index("s")
        zrows = ACC // _NS
        pltpu.sync_copy(z_hbm.at[pl.ds(s * zrows, zrows)],
                        acc.at[pl.ds(s * zrows, zrows)])
        plsc.subcore_barrier()

        def unpack(t, idx):
            row = pk_v.at[t]
            for j in range(0, _LANES, 16):
                v = row[pl.ds(j, 16)]
                idx.at[0][pl.ds(j, 16)] = v & 0xFFFF
                idx.at[1][pl.ds(j, 16)] = v >> 16

        def run(pk_hbm, K):
            pltpu.sync_copy(pk_hbm.at[s], pk_v.at[pl.ds(0, K)])
            unpack(0, idx_a)
            pltpu.async_copy(h_hbm.at[idx_a.at[0]], rows_a, sem_a)

            @pl.loop(0, K - 1, step=2)
            def _(t):
                unpack(t + 1, idx_b)
                pltpu.async_copy(h_hbm.at[idx_b.at[0]], rows_b, sem_b)
                pltpu.make_async_copy(h_hbm.at[idx_a.at[0]], rows_a,
                                      sem_a).wait()
                pltpu.sync_copy(rows_a, acc.at[idx_a.at[1]], add=True)
                unpack(t + 2, idx_a)
                pltpu.async_copy(h_hbm.at[idx_a.at[0]], rows_a, sem_a)
                pltpu.make_async_copy(h_hbm.at[idx_b.at[0]], rows_b,
                                      sem_b).wait()
                pltpu.sync_copy(rows_b, acc.at[idx_b.at[1]], add=True)

            pltpu.make_async_copy(h_hbm.at[idx_a.at[0]], rows_a, sem_a).wait()
            pltpu.sync_copy(rows_a, acc.at[idx_a.at[1]], add=True)

        @pl.when(c == 0)
        def _():
            run(pk0_hbm, K0)

        @pl.when(c == 1)
        def _():
            run(pk1_hbm, K1)

        plsc.subcore_barrier()
        pltpu.sync_copy(acc.at[pl.ds(s * zrows, zrows)],
                        out_hbm.at[c, pl.ds(s * zrows, zrows)])

    return k(h, pk0, pk1, zeros)


_DN = (((1,), (1,)), ((), ()))


def _tc_self(h, W_self, bias2):
    B = 1000

    def body(h_ref, ws_ref, b_ref, o_ref):
        o_ref[...] = lax.dot_general(
            h_ref[...], ws_ref[...], _DN,
            preferred_element_type=jnp.float32,
            precision=lax.Precision.HIGHEST) + b_ref[...]

    return pl.pallas_call(
        body,
        grid=(_N // B,),
        in_specs=[
            pl.BlockSpec((B, _D), lambda i: (i, 0)),
            pl.BlockSpec((_D, _D), lambda i: (0, 0)),
            pl.BlockSpec((1, _D), lambda i: (0, 0)),
        ],
        out_specs=pl.BlockSpec((B, _D), lambda i: (i, 0)),
        out_shape=jax.ShapeDtypeStruct((_N, _D), jnp.float32),
    )(h, W_self, bias2)


def _tc_out(selft, p, deg2, W_neigh):
    B = 1000

    def body(st_ref, p0_ref, p1_ref, deg_ref, wn_ref, o_ref):
        a = (p0_ref[0] + p1_ref[0]) / deg_ref[...]
        acc = st_ref[...] + lax.dot_general(
            a, wn_ref[...], _DN, preferred_element_type=jnp.float32,
            precision=lax.Precision.HIGHEST)
        o_ref[...] = jnp.maximum(acc, 0.0)

    return pl.pallas_call(
        body,
        grid=(_N // B,),
        in_specs=[
            pl.BlockSpec((B, _D), lambda i: (i, 0)),
            pl.BlockSpec((1, B, _D), lambda i: (0, i, 0)),
            pl.BlockSpec((1, B, _D), lambda i: (1, i, 0)),
            pl.BlockSpec((B, 1), lambda i: (i, 0)),
            pl.BlockSpec((_D, _D), lambda i: (0, 0)),
        ],
        out_specs=pl.BlockSpec((B, _D), lambda i: (i, 0)),
        out_shape=jax.ShapeDtypeStruct((_N, _D), jnp.float32),
    )(selft, p, p, deg2, W_neigh)


def kernel(h, edge_index, deg, W_self, W_neigh, bias):
    src = edge_index[0]
    dst = edge_index[1]
    K0, K1 = 147, 33
    e0 = _NS * K0 * _LANES
    e1_pad = _NS * K1 * _LANES
    assert e0 + e1_pad >= _E
    acc_rows = 128 * (-(-(_N + 1) // 128))
    packed = src | (dst << 16)
    pad = jnp.full((e0 + e1_pad - _E,), _N << 16, jnp.int32)
    pk0 = packed[:e0].reshape(_NS, K0, _LANES)
    pk1 = jnp.concatenate([packed[e0:], pad]).reshape(_NS, K1, _LANES)
    zeros = jnp.zeros((acc_rows, _D), jnp.float32)
    selft = _tc_self(h, W_self, bias.reshape(1, _D))
    partials = _sc_segment_sum(h, pk0, pk1, zeros)
    return _tc_out(selft, partials, deg.reshape(_N, 1), W_neigh)


# speedup vs baseline: 1.0505x; 1.0031x over previous
import functools

import jax
import jax.numpy as jnp
from jax import lax
from jax.experimental import pallas as pl
from jax.experimental.pallas import tpu as pltpu
from jax.experimental.pallas import tpu_sc as plsc

_N = 10000
_E = 320000
_D = 128
_NC = 2
_NS = 16
_NW = _NC * _NS
_LANES = 112


def _sc_segment_sum(h, pk0, pk1, zeros):
    K0 = pk0.shape[1]
    K1 = pk1.shape[1]
    assert K0 % 2 == 1 and K1 % 2 == 1, "pipeline assumes odd chunk counts"
    ACC = zeros.shape[0]
    mesh = plsc.VectorSubcoreMesh(core_axis_name="c", subcore_axis_name="s")

    @functools.partial(
        pl.kernel,
        out_type=jax.ShapeDtypeStruct((_NC, ACC, _D), jnp.float32),
        mesh=mesh,
        scratch_types=[
            pltpu.VMEM((max(K0, K1), _LANES), jnp.int32),
            pltpu.VMEM((2, _LANES), jnp.int32),
            pltpu.VMEM((2, _LANES), jnp.int32),
            pltpu.VMEM((_LANES, _D), jnp.float32),
            pltpu.VMEM((_LANES, _D), jnp.float32),
            pltpu.VMEM_SHARED((ACC, _D), jnp.float32),
            pltpu.SemaphoreType.DMA,
            pltpu.SemaphoreType.DMA,
        ],
    )
    def k(h_hbm, pk0_hbm, pk1_hbm, z_hbm, out_hbm, pk_v, idx_a, idx_b,
          rows_a, rows_b, acc, sem_a, sem_b):
        c = lax.axis_index("c")
        s = lax.axis_index("s")
        zrows = ACC // _NS
        pltpu.sync_copy(z_hbm.at[pl.ds(s * zrows, zrows)],
                        acc.at[pl.ds(s * zrows, zrows)])
        plsc.subcore_barrier()

        def unpack(t, idx):
            row = pk_v.at[t]
            for j in range(0, _LANES, 16):
                v = row[pl.ds(j, 16)]
                idx.at[0][pl.ds(j, 16)] = v & 0xFFFF
                idx.at[1][pl.ds(j, 16)] = v >> 16

        def run(pk_hbm, K):
            pltpu.sync_copy(pk_hbm.at[s], pk_v.at[pl.ds(0, K)])
            unpack(0, idx_a)
            pltpu.async_copy(h_hbm.at[idx_a.at[0]], rows_a, sem_a)

            @pl.loop(0, K - 1, step=2)
            def _(t):
                unpack(t + 1, idx_b)
                pltpu.async_copy(h_hbm.at[idx_b.at[0]], rows_b, sem_b)
                pltpu.make_async_copy(h_hbm.at[idx_a.at[0]], rows_a,
                                      sem_a).wait()
                pltpu.sync_copy(rows_a, acc.at[idx_a.at[1]], add=True)
                unpack(t + 2, idx_a)
                pltpu.async_copy(h_hbm.at[idx_a.at[0]], rows_a, sem_a)
                pltpu.make_async_copy(h_hbm.at[idx_b.at[0]], rows_b,
                                      sem_b).wait()
                pltpu.sync_copy(rows_b, acc.at[idx_b.at[1]], add=True)

            pltpu.make_async_copy(h_hbm.at[idx_a.at[0]], rows_a, sem_a).wait()
            pltpu.sync_copy(rows_a, acc.at[idx_a.at[1]], add=True)

        @pl.when(c == 0)
        def _():
            run(pk0_hbm, K0)

        @pl.when(c == 1)
        def _():
            run(pk1_hbm, K1)

        plsc.subcore_barrier()
        pltpu.sync_copy(acc.at[pl.ds(s * zrows, zrows)],
                        out_hbm.at[c, pl.ds(s * zrows, zrows)])

    return k(h, pk0, pk1, zeros)


_DN = (((1,), (1,)), ((), ()))


def _tc_self(h, W_self, bias2):
    B = 1000

    def body(h_ref, ws_ref, b_ref, o_ref):
        o_ref[...] = lax.dot_general(
            h_ref[...], ws_ref[...], _DN,
            preferred_element_type=jnp.float32,
            precision=lax.Precision.HIGHEST) + b_ref[...]

    return pl.pallas_call(
        body,
        grid=(_N // B,),
        in_specs=[
            pl.BlockSpec((B, _D), lambda i: (i, 0)),
            pl.BlockSpec((_D, _D), lambda i: (0, 0)),
            pl.BlockSpec((1, _D), lambda i: (0, 0)),
        ],
        out_specs=pl.BlockSpec((B, _D), lambda i: (i, 0)),
        out_shape=jax.ShapeDtypeStruct((_N, _D), jnp.float32),
    )(h, W_self, bias2)


def _tc_out(selft, p, deg2, W_neigh):
    B = 1000

    def body(st_ref, p0_ref, p1_ref, deg_ref, wn_ref, o_ref):
        a = (p0_ref[0] + p1_ref[0]) / deg_ref[...]
        acc = st_ref[...] + lax.dot_general(
            a, wn_ref[...], _DN, preferred_element_type=jnp.float32,
            precision=lax.Precision.HIGHEST)
        o_ref[...] = jnp.maximum(acc, 0.0)

    return pl.pallas_call(
        body,
        grid=(_N // B,),
        in_specs=[
            pl.BlockSpec((B, _D), lambda i: (i, 0)),
            pl.BlockSpec((1, B, _D), lambda i: (0, i, 0)),
            pl.BlockSpec((1, B, _D), lambda i: (1, i, 0)),
            pl.BlockSpec((B, 1), lambda i: (i, 0)),
            pl.BlockSpec((_D, _D), lambda i: (0, 0)),
        ],
        out_specs=pl.BlockSpec((B, _D), lambda i: (i, 0)),
        out_shape=jax.ShapeDtypeStruct((_N, _D), jnp.float32),
    )(selft, p, p, deg2, W_neigh)


def kernel(h, edge_index, deg, W_self, W_neigh, bias):
    src = edge_index[0]
    dst = edge_index[1]
    K0, K1 = 149, 31
    e0 = _NS * K0 * _LANES
    e1_pad = _NS * K1 * _LANES
    assert e0 + e1_pad >= _E
    acc_rows = 128 * (-(-(_N + 1) // 128))
    packed = src | (dst << 16)
    pad = jnp.full((e0 + e1_pad - _E,), _N << 16, jnp.int32)
    pk0 = packed[:e0].reshape(_NS, K0, _LANES)
    pk1 = jnp.concatenate([packed[e0:], pad]).reshape(_NS, K1, _LANES)
    zeros = jnp.zeros((acc_rows, _D), jnp.float32)
    selft = _tc_self(h, W_self, bias.reshape(1, _D))
    partials = _sc_segment_sum(h, pk0, pk1, zeros)
    return _tc_out(selft, partials, deg.reshape(_N, 1), W_neigh)
